# Initial kernel scaffold; baseline (speedup 1.0000x reference)
import functools

import jax
import jax.numpy as jnp
from jax import lax
from jax.experimental import pallas as pl
from jax.experimental.pallas import tpu as pltpu
from jax.experimental.pallas import tpu_sc as plsc

VOCAB_SIZE = 1000000
EMBED_DIM = 64
SEQ_LEN = 200
BATCH = 1024

ROWS = BATCH * SEQ_LEN
NUM_WORKERS = 32
ROWS_PER_WORKER = ROWS // NUM_WORKERS
CHUNK = 2 * SEQ_LEN
NUM_CHUNKS = ROWS_PER_WORKER // CHUNK
LANES = 16
VPR = EMBED_DIM // LANES


def _body(x_hbm, tok_hbm, pos_hbm, out_hbm, idx_v, buf_v, pos_v, sem):
    wid = lax.axis_index("s") * 2 + lax.axis_index("c")
    base0 = wid * ROWS_PER_WORKER

    pltpu.sync_copy(pos_hbm, pos_v)

    def chunk_body(c, carry):
        base = base0 + c * CHUNK
        pltpu.sync_copy(x_hbm.at[pl.ds(base, CHUNK)], idx_v)
        pltpu.async_copy(tok_hbm.at[idx_v], buf_v, sem).wait()

        def row_body(r, carry2):
            p = lax.rem(r, SEQ_LEN)
            for j in range(VPR):
                sl = pl.ds(j * LANES, LANES)
                buf_v[r, sl] = buf_v[r, sl] + pos_v[p, sl]
            return carry2

        lax.fori_loop(0, CHUNK, row_body, 0, unroll=2)
        pltpu.sync_copy(buf_v, out_hbm.at[pl.ds(base, CHUNK)])
        return carry

    lax.fori_loop(0, NUM_CHUNKS, chunk_body, 0)


def kernel(x, token_table, pos_table):
    xf = x.reshape(ROWS).astype(jnp.int32)
    mesh = plsc.VectorSubcoreMesh(core_axis_name="c", subcore_axis_name="s")
    run = functools.partial(
        pl.kernel,
        mesh=mesh,
        out_type=jax.ShapeDtypeStruct((ROWS, EMBED_DIM), jnp.float32),
        scratch_types=[
            pltpu.VMEM((CHUNK,), jnp.int32),
            pltpu.VMEM((CHUNK, EMBED_DIM), jnp.float32),
            pltpu.VMEM((SEQ_LEN, EMBED_DIM), jnp.float32),
            pltpu.SemaphoreType.DMA,
        ],
    )(_body)
    out = run(xf, token_table, pos_table)
    return out.reshape(BATCH, SEQ_LEN, EMBED_DIM)


# speedup vs baseline: 1.1703x; 1.1703x over previous
import functools

import jax
import jax.numpy as jnp
from jax import lax
from jax.experimental import pallas as pl
from jax.experimental.pallas import tpu as pltpu
from jax.experimental.pallas import tpu_sc as plsc

VOCAB_SIZE = 1000000
EMBED_DIM = 64
SEQ_LEN = 200
BATCH = 1024

ROWS = BATCH * SEQ_LEN
NUM_WORKERS = 32
ROWS_PER_WORKER = ROWS // NUM_WORKERS
CHUNK = 2 * SEQ_LEN
NUM_CHUNKS = ROWS_PER_WORKER // CHUNK
LANES = 16
VPR = EMBED_DIM // LANES


def _body(x_hbm, tok_hbm, pos_hbm, out_hbm, idx_v, buf_v, pos_v, sem):
    wid = lax.axis_index("s") * 2 + lax.axis_index("c")
    base0 = wid * ROWS_PER_WORKER

    pltpu.sync_copy(pos_hbm, pos_v)

    def chunk_body(c, carry):
        base = base0 + c * CHUNK
        pltpu.sync_copy(x_hbm.at[pl.ds(base, CHUNK)], idx_v)
        pltpu.async_copy(tok_hbm.at[idx_v], buf_v, sem).wait()

        def row_body(r, carry2):
            p = lax.rem(r, SEQ_LEN)
            for j in range(VPR):
                sl = pl.ds(j * LANES, LANES)
                buf_v[r, sl] = buf_v[r, sl] + pos_v[p, sl]
            return carry2

        lax.fori_loop(0, CHUNK, row_body, 0, unroll=2)
        pltpu.sync_copy(buf_v, out_hbm.at[pl.ds(base, CHUNK)])
        return carry

    lax.fori_loop(0, NUM_CHUNKS, chunk_body, 0)


def kernel(x, token_table, pos_table):
    xf = x.reshape(ROWS).astype(jnp.int32)
    mesh = plsc.VectorSubcoreMesh(core_axis_name="c", subcore_axis_name="s")
    run = functools.partial(
        pl.kernel,
        mesh=mesh,
        out_type=jax.ShapeDtypeStruct((ROWS, EMBED_DIM), jnp.float32),
        scratch_types=[
            pltpu.VMEM((CHUNK,), jnp.int32),
            pltpu.VMEM((CHUNK, EMBED_DIM), jnp.float32),
            pltpu.VMEM((SEQ_LEN, EMBED_DIM), jnp.float32),
            pltpu.SemaphoreType.DMA,
        ],
        compiler_params=pltpu.CompilerParams(use_tc_tiling_on_sc=False),
    )(_body)
    out = run(xf, token_table, pos_table)
    return out.reshape(BATCH, SEQ_LEN, EMBED_DIM)


# speedup vs baseline: 1.3346x; 1.1404x over previous
import functools

import jax
import jax.numpy as jnp
from jax import lax
from jax.experimental import pallas as pl
from jax.experimental.pallas import tpu as pltpu
from jax.experimental.pallas import tpu_sc as plsc

VOCAB_SIZE = 1000000
EMBED_DIM = 64
SEQ_LEN = 200
BATCH = 1024

ROWS = BATCH * SEQ_LEN
NUM_WORKERS = 32
ROWS_PER_WORKER = ROWS // NUM_WORKERS
SEQ_PER_CHUNK = 2
CHUNK = SEQ_PER_CHUNK * SEQ_LEN
NUM_CHUNKS = ROWS_PER_WORKER // CHUNK
LANES = 16
VPR = EMBED_DIM // LANES


def _body(x_hbm, tok_hbm, pos_hbm, out_hbm,
          idx_a, idx_b, buf_a, buf_b, pos_v,
          gsem_a, gsem_b, ssem_a, ssem_b):
    wid = lax.axis_index("s") * 2 + lax.axis_index("c")
    base0 = wid * ROWS_PER_WORKER

    idx_v = (idx_a, idx_b)
    buf_v = (buf_a, buf_b)
    gsem = (gsem_a, gsem_b)
    ssem = (ssem_a, ssem_b)

    pltpu.sync_copy(pos_hbm, pos_v)

    gather = [None, None]
    store = [None, None]

    def launch(c):
        b = c % 2
        pltpu.sync_copy(x_hbm.at[pl.ds(base0 + c * CHUNK, CHUNK)], idx_v[b])
        gather[b] = pltpu.async_copy(tok_hbm.at[idx_v[b]], buf_v[b], gsem[b])

    launch(0)
    for c in range(NUM_CHUNKS):
        b = c % 2
        nb = (c + 1) % 2
        if c + 1 < NUM_CHUNKS:
            if store[nb] is not None:
                store[nb].wait()
            launch(c + 1)
        gather[b].wait()

        def row_body(r, carry):
            for j in range(VPR):
                sl = pl.ds(j * LANES, LANES)
                plsc.addupdate(buf_v[b].at[r, sl], pos_v[r, sl])
            return carry

        lax.fori_loop(0, CHUNK, row_body, 0, unroll=4)
        store[b] = pltpu.async_copy(
            buf_v[b], out_hbm.at[pl.ds(base0 + c * CHUNK, CHUNK)], ssem[b])

    store[(NUM_CHUNKS - 1) % 2].wait()
    store[NUM_CHUNKS % 2].wait()


def kernel(x, token_table, pos_table):
    xf = x.reshape(ROWS).astype(jnp.int32)
    pos_tiled = jnp.tile(pos_table, (SEQ_PER_CHUNK, 1))
    mesh = plsc.VectorSubcoreMesh(core_axis_name="c", subcore_axis_name="s")
    run = functools.partial(
        pl.kernel,
        mesh=mesh,
        out_type=jax.ShapeDtypeStruct((ROWS, EMBED_DIM), jnp.float32),
        scratch_types=[
            pltpu.VMEM((CHUNK,), jnp.int32),
            pltpu.VMEM((CHUNK,), jnp.int32),
            pltpu.VMEM((CHUNK, EMBED_DIM), jnp.float32),
            pltpu.VMEM((CHUNK, EMBED_DIM), jnp.float32),
            pltpu.VMEM((CHUNK, EMBED_DIM), jnp.float32),
            pltpu.SemaphoreType.DMA,
            pltpu.SemaphoreType.DMA,
            pltpu.SemaphoreType.DMA,
            pltpu.SemaphoreType.DMA,
        ],
        compiler_params=pltpu.CompilerParams(use_tc_tiling_on_sc=False),
    )(_body)
    out = run(xf, token_table, pos_tiled)
    return out.reshape(BATCH, SEQ_LEN, EMBED_DIM)
